# Initial kernel scaffold; baseline (speedup 1.0000x reference)
import functools

import jax
import jax.numpy as jnp
from jax import lax
from jax.experimental import pallas as pl
from jax.experimental.pallas import tpu as pltpu
from jax.experimental.pallas import tpu_sc as plsc

_B, _N, _C, _H, _W = 2, 4, 96, 224, 224
_L = 16
_CC = 16
_NUM_WORKERS = 32
_UNITS_PER_WORKER = (_B * _H) // _NUM_WORKERS
_GROUPS = _W // _L
_CHUNKS = _C // _CC


def _rank_and_sources(ts_v, src_v):
    for g in range(_GROUPS):
        sl = pl.ds(g * _L, _L)
        t = [ts_v[m, sl] for m in range(_N)]
        ranks = []
        for m in range(_N):
            r = jnp.zeros((_L,), jnp.int32)
            for j in range(_N):
                if j == m:
                    continue
                gt = (t[j] > t[m]).astype(jnp.int32)
                if j < m:
                    eq = (t[j] == t[m]).astype(jnp.int32)
                    r = r + gt + eq
                else:
                    r = r + gt
            ranks.append(r)
        for n in range(_N):
            s = jnp.zeros((_L,), jnp.int32)
            for m in range(1, _N):
                s = s + jnp.where(ranks[m] == n, m, 0)
            src_v[n, sl] = s


def _body(feat_hbm, ts_hbm, out_hbm, ts_v, src_v, feat_v, out_v):
    wid = lax.axis_index("s") * 2 + lax.axis_index("c")
    lane = lax.broadcasted_iota(jnp.int32, (_L,), 0)

    def unit_body(u, _):
        unit = wid * _UNITS_PER_WORKER + u
        b = unit // _H
        h = unit % _H

        pltpu.sync_copy(ts_hbm.at[b, :, h, :], ts_v)
        _rank_and_sources(ts_v, src_v)

        def chunk_body(cc, _):
            c0 = cc * _CC
            pltpu.sync_copy(feat_hbm.at[b, :, pl.ds(c0, _CC), h, :], feat_v)

            def chan_body(c, _):
                c_splat = jnp.full((_L,), c, jnp.int32)
                for g in range(_GROUPS):
                    sl = pl.ds(g * _L, _L)
                    p16 = lane + g * _L
                    for n in range(_N):
                        s = src_v[n, sl]
                        out_v[n, c, sl] = plsc.load_gather(
                            feat_v, [s, c_splat, p16])
                return 0

            lax.fori_loop(0, _CC, chan_body, 0)
            pltpu.sync_copy(out_v, out_hbm.at[b, :, pl.ds(c0, _CC), h, :])
            return 0

        lax.fori_loop(0, _CHUNKS, chunk_body, 0)
        return 0

    lax.fori_loop(0, _UNITS_PER_WORKER, unit_body, 0)


@functools.partial(jax.jit, static_argnames=("dim",))
def kernel(features, timestamps, dim):
    del dim
    mesh = plsc.VectorSubcoreMesh(core_axis_name="c", subcore_axis_name="s")
    run = pl.kernel(
        _body,
        out_type=jax.ShapeDtypeStruct(features.shape, features.dtype),
        mesh=mesh,
        scratch_types=[
            pltpu.VMEM((_N, _W), jnp.float32),
            pltpu.VMEM((_N, _W), jnp.int32),
            pltpu.VMEM((_N, _CC, _W), jnp.float32),
            pltpu.VMEM((_N, _CC, _W), jnp.float32),
        ],
    )
    return run(features, timestamps)


# speedup vs baseline: 3.3548x; 3.3548x over previous
import functools

import jax
import jax.numpy as jnp
from jax import lax
from jax.experimental import pallas as pl
from jax.experimental.pallas import tpu as pltpu
from jax.experimental.pallas import tpu_sc as plsc

_B, _N, _C, _H, _W = 2, 4, 96, 224, 224
_L = 16
_CC = 16
_NUM_WORKERS = 32
_UNITS_PER_WORKER = (_B * _H) // _NUM_WORKERS
_GROUPS = _W // _L
_CHUNKS = _C // _CC


def _rank_and_sources(ts_v, src_v, consts):
    zero = consts[0]
    one = consts[1]
    for g in range(_GROUPS):
        sl = pl.ds(g * _L, _L)
        t = [ts_v[m, sl] for m in range(_N)]
        ranks = []
        for m in range(_N):
            r = zero
            for j in range(_N):
                if j == m:
                    continue
                r = r + jnp.where(t[j] > t[m], one, zero)
                if j < m:
                    r = r + jnp.where(t[j] == t[m], one, zero)
            ranks.append(r)
        for n in range(_N):
            s = zero
            for m in range(1, _N):
                s = s + jnp.where(ranks[m] == consts[n], consts[m], zero)
            src_v[n, sl] = s


def _body(feat_hbm, ts_hbm, out_hbm, ts_v, src_v, feat_v, out_v):
    wid = lax.axis_index("s") * 2 + lax.axis_index("c")
    consts = [jnp.full((_L,), m, jnp.int32) for m in range(_N)]

    def unit_body(u, _):
        unit = wid * _UNITS_PER_WORKER + u
        b = unit // _H
        h = unit % _H

        pltpu.sync_copy(ts_hbm.at[b, :, h, :], ts_v)
        _rank_and_sources(ts_v, src_v, consts)

        def chunk_body(cc, _):
            c0 = cc * _CC
            pltpu.sync_copy(feat_hbm.at[b, :, pl.ds(c0, _CC), h, :], feat_v)

            for g in range(_GROUPS):
                sl = pl.ds(g * _L, _L)
                srcs = [src_v[n, sl] for n in range(_N)]
                masks = [[srcs[n] == consts[m] for m in range(_N - 1)]
                         for n in range(_N)]

                def chan_body(c, _, sl=sl, masks=masks):
                    f = [feat_v[m, c, sl] for m in range(_N)]
                    for n in range(_N):
                        v = f[_N - 1]
                        for m in range(_N - 2, -1, -1):
                            v = jnp.where(masks[n][m], f[m], v)
                        out_v[n, c, sl] = v
                    return 0

                lax.fori_loop(0, _CC, chan_body, 0)
            pltpu.sync_copy(out_v, out_hbm.at[b, :, pl.ds(c0, _CC), h, :])
            return 0

        lax.fori_loop(0, _CHUNKS, chunk_body, 0)
        return 0

    lax.fori_loop(0, _UNITS_PER_WORKER, unit_body, 0)


def kernel(features, timestamps, dim):
    del dim
    mesh = plsc.VectorSubcoreMesh(core_axis_name="c", subcore_axis_name="s")
    run = pl.kernel(
        _body,
        out_type=jax.ShapeDtypeStruct(features.shape, features.dtype),
        mesh=mesh,
        compiler_params=pltpu.CompilerParams(use_tc_tiling_on_sc=False),
        scratch_types=[
            pltpu.VMEM((_N, _W), jnp.float32),
            pltpu.VMEM((_N, _W), jnp.int32),
            pltpu.VMEM((_N, _CC, _W), jnp.float32),
            pltpu.VMEM((_N, _CC, _W), jnp.float32),
        ],
    )
    return run(features, timestamps)


# speedup vs baseline: 3.9405x; 1.1746x over previous
import functools

import jax
import jax.numpy as jnp
from jax import lax
from jax.experimental import pallas as pl
from jax.experimental.pallas import tpu as pltpu
from jax.experimental.pallas import tpu_sc as plsc

_B, _N, _C, _H, _W = 2, 4, 96, 224, 224
_L = 16
_CC = 32
_NUM_WORKERS = 32
_UNITS_PER_WORKER = (_B * _H) // _NUM_WORKERS
_GROUPS = _W // _L
_CHUNKS = _C // _CC


def _rank_and_sources(ts_v, src_v, consts):
    zero = consts[0]
    one = consts[1]
    for g in range(_GROUPS):
        sl = pl.ds(g * _L, _L)
        t = [ts_v[m, sl] for m in range(_N)]
        ranks = []
        for m in range(_N):
            r = zero
            for j in range(_N):
                if j == m:
                    continue
                r = r + jnp.where(t[j] > t[m], one, zero)
                if j < m:
                    r = r + jnp.where(t[j] == t[m], one, zero)
            ranks.append(r)
        for n in range(_N):
            s = zero
            for m in range(1, _N):
                s = s + jnp.where(ranks[m] == consts[n], consts[m], zero)
            src_v[n, sl] = s


def _permute_chunk(feat_b, out_b, src_v, consts):
    for g in range(_GROUPS):
        sl = pl.ds(g * _L, _L)
        srcs = [src_v[n, sl] for n in range(_N)]
        masks = [[srcs[n] == consts[m] for m in range(_N - 1)]
                 for n in range(_N)]

        def chan_body(c, _, sl=sl, masks=masks):
            f = [feat_b[m, c, sl] for m in range(_N)]
            for n in range(_N):
                v = f[_N - 1]
                for m in range(_N - 2, -1, -1):
                    v = jnp.where(masks[n][m], f[m], v)
                out_b[n, c, sl] = v
            return 0

        lax.fori_loop(0, _CC, chan_body, 0)


def _body(feat_hbm, ts_hbm, out_hbm, ts_v, src_v, feat_v, out_v,
          in_sems, out_sems):
    wid = lax.axis_index("s") * 2 + lax.axis_index("c")
    consts = [jnp.full((_L,), m, jnp.int32) for m in range(_N)]

    def unit_body(u, _):
        unit = wid * _UNITS_PER_WORKER + u
        b = unit // _H
        h = unit % _H

        pltpu.sync_copy(ts_hbm.at[b, :, h, :], ts_v)
        _rank_and_sources(ts_v, src_v, consts)

        def in_copy(cc):
            buf = cc % 2
            return pltpu.make_async_copy(
                feat_hbm.at[b, :, pl.ds(cc * _CC, _CC), h, :],
                feat_v.at[buf], in_sems.at[buf])

        def out_copy(cc):
            buf = cc % 2
            return pltpu.make_async_copy(
                out_v.at[buf],
                out_hbm.at[b, :, pl.ds(cc * _CC, _CC), h, :],
                out_sems.at[buf])

        in_copy(0).start()
        for cc in range(_CHUNKS):
            buf = cc % 2
            if cc + 1 < _CHUNKS:
                in_copy(cc + 1).start()
            in_copy(cc).wait()
            if cc >= 2:
                out_copy(cc - 2).wait()
            _permute_chunk(feat_v.at[buf], out_v.at[buf], src_v, consts)
            out_copy(cc).start()
        for cc in range(max(_CHUNKS - 2, 0), _CHUNKS):
            out_copy(cc).wait()
        return 0

    lax.fori_loop(0, _UNITS_PER_WORKER, unit_body, 0)


def kernel(features, timestamps, dim):
    del dim
    mesh = plsc.VectorSubcoreMesh(core_axis_name="c", subcore_axis_name="s")
    run = pl.kernel(
        _body,
        out_type=jax.ShapeDtypeStruct(features.shape, features.dtype),
        mesh=mesh,
        compiler_params=pltpu.CompilerParams(use_tc_tiling_on_sc=False),
        scratch_types=[
            pltpu.VMEM((_N, _W), jnp.float32),
            pltpu.VMEM((_N, _W), jnp.int32),
            pltpu.VMEM((2, _N, _CC, _W), jnp.float32),
            pltpu.VMEM((2, _N, _CC, _W), jnp.float32),
            pltpu.SemaphoreType.DMA((2,)),
            pltpu.SemaphoreType.DMA((2,)),
        ],
    )
    return run(features, timestamps)


# speedup vs baseline: 4.1792x; 1.0606x over previous
import functools

import jax
import jax.numpy as jnp
from jax import lax
from jax.experimental import pallas as pl
from jax.experimental.pallas import tpu as pltpu
from jax.experimental.pallas import tpu_sc as plsc

_B, _N, _C, _H, _W = 2, 4, 96, 224, 224
_L = 16
_HH = 8
_CQ = 24
_CC = 4
_CHUNKS = _CQ // _CC
_NUM_WORKERS = 32
_HB = _H // _HH
_UNITS = _B * _HB * (_C // _CQ)
_UNITS_PER_WORKER = _UNITS // _NUM_WORKERS
_GROUPS = _HH * (_W // _L)


def _rank_and_sources(ts_v, src_v, consts):
    zero = consts[0]
    one = consts[1]

    def group_body(fg, _):
        hh = fg // (_W // _L)
        sl = pl.ds((fg % (_W // _L)) * _L, _L)
        t = [ts_v[m, hh, sl] for m in range(_N)]
        ranks = []
        for m in range(_N):
            r = zero
            for j in range(_N):
                if j == m:
                    continue
                r = r + jnp.where(t[j] > t[m], one, zero)
                if j < m:
                    r = r + jnp.where(t[j] == t[m], one, zero)
            ranks.append(r)
        for n in range(_N):
            s = zero
            for m in range(1, _N):
                s = s + jnp.where(ranks[m] == consts[n], consts[m], zero)
            src_v[n, hh, sl] = s
        return 0

    lax.fori_loop(0, _GROUPS, group_body, 0)


def _permute_chunk(feat_b, out_b, src_v, consts):
    def group_body(fg, _):
        hh = fg // (_W // _L)
        sl = pl.ds((fg % (_W // _L)) * _L, _L)
        srcs = [src_v[n, hh, sl] for n in range(_N)]
        masks = [[srcs[n] == consts[m] for m in range(_N - 1)]
                 for n in range(_N)]
        for c in range(_CC):
            f = [feat_b[m, c, hh, sl] for m in range(_N)]
            for n in range(_N):
                v = f[_N - 1]
                for m in range(_N - 2, -1, -1):
                    v = jnp.where(masks[n][m], f[m], v)
                out_b[n, c, hh, sl] = v
        return 0

    lax.fori_loop(0, _GROUPS, group_body, 0)


def _body(feat_hbm, ts_hbm, out_hbm, ts_v, src_v, feat_v, out_v,
          in_sems, out_sems):
    wid = lax.axis_index("s") * 2 + lax.axis_index("c")
    consts = [jnp.full((_L,), m, jnp.int32) for m in range(_N)]

    def unit_body(u, _):
        t = wid * _UNITS_PER_WORKER + u
        b = t // (_HB * (_C // _CQ))
        r = t % (_HB * (_C // _CQ))
        h0 = (r // (_C // _CQ)) * _HH
        c0 = (r % (_C // _CQ)) * _CQ

        pltpu.sync_copy(ts_hbm.at[b, :, pl.ds(h0, _HH), :], ts_v)
        _rank_and_sources(ts_v, src_v, consts)

        def in_copy(cc):
            buf = cc % 2
            return pltpu.make_async_copy(
                feat_hbm.at[b, :, pl.ds(c0 + cc * _CC, _CC),
                            pl.ds(h0, _HH), :],
                feat_v.at[buf], in_sems.at[buf])

        def out_copy(cc):
            buf = cc % 2
            return pltpu.make_async_copy(
                out_v.at[buf],
                out_hbm.at[b, :, pl.ds(c0 + cc * _CC, _CC),
                           pl.ds(h0, _HH), :],
                out_sems.at[buf])

        in_copy(0).start()
        for cc in range(_CHUNKS):
            buf = cc % 2
            if cc + 1 < _CHUNKS:
                in_copy(cc + 1).start()
            in_copy(cc).wait()
            if cc >= 2:
                out_copy(cc - 2).wait()
            _permute_chunk(feat_v.at[buf], out_v.at[buf], src_v, consts)
            out_copy(cc).start()
        for cc in range(max(_CHUNKS - 2, 0), _CHUNKS):
            out_copy(cc).wait()
        return 0

    lax.fori_loop(0, _UNITS_PER_WORKER, unit_body, 0)


def kernel(features, timestamps, dim):
    del dim
    mesh = plsc.VectorSubcoreMesh(core_axis_name="c", subcore_axis_name="s")
    run = pl.kernel(
        _body,
        out_type=jax.ShapeDtypeStruct(features.shape, features.dtype),
        mesh=mesh,
        compiler_params=pltpu.CompilerParams(use_tc_tiling_on_sc=False),
        scratch_types=[
            pltpu.VMEM((_N, _HH, _W), jnp.float32),
            pltpu.VMEM((_N, _HH, _W), jnp.int32),
            pltpu.VMEM((2, _N, _CC, _HH, _W), jnp.float32),
            pltpu.VMEM((2, _N, _CC, _HH, _W), jnp.float32),
            pltpu.SemaphoreType.DMA((2,)),
            pltpu.SemaphoreType.DMA((2,)),
        ],
    )
    return run(features, timestamps)


# speedup vs baseline: 4.1979x; 1.0045x over previous
import functools

import jax
import jax.numpy as jnp
from jax import lax
from jax.experimental import pallas as pl
from jax.experimental.pallas import tpu as pltpu
from jax.experimental.pallas import tpu_sc as plsc

_B, _N, _C, _H, _W = 2, 4, 96, 224, 224
_L = 16
_HH = 8
_CQ = 24
_CC = 2
_CHUNKS = _CQ // _CC
_NBUF = 4
_DEPTH = 3
_NUM_WORKERS = 32
_HB = _H // _HH
_UNITS = _B * _HB * (_C // _CQ)
_UNITS_PER_WORKER = _UNITS // _NUM_WORKERS
_GROUPS = _HH * (_W // _L)


def _rank_and_sources(ts_v, src_v, consts):
    zero = consts[0]
    one = consts[1]

    def group_body(fg, _):
        hh = fg // (_W // _L)
        sl = pl.ds((fg % (_W // _L)) * _L, _L)
        t = [ts_v[m, hh, sl] for m in range(_N)]
        ranks = []
        for m in range(_N):
            r = zero
            for j in range(_N):
                if j == m:
                    continue
                r = r + jnp.where(t[j] > t[m], one, zero)
                if j < m:
                    r = r + jnp.where(t[j] == t[m], one, zero)
            ranks.append(r)
        for n in range(_N):
            s = zero
            for m in range(1, _N):
                s = s + jnp.where(ranks[m] == consts[n], consts[m], zero)
            src_v[n, hh, sl] = s
        return 0

    lax.fori_loop(0, _GROUPS, group_body, 0)


def _permute_chunk(feat_b, out_b, src_v, consts):
    def group_body(fg, _):
        hh = fg // (_W // _L)
        sl = pl.ds((fg % (_W // _L)) * _L, _L)
        srcs = [src_v[n, hh, sl] for n in range(_N)]
        masks = [[srcs[n] == consts[m] for m in range(_N - 1)]
                 for n in range(_N)]
        for c in range(_CC):
            f = [feat_b[m, c, hh, sl] for m in range(_N)]
            for n in range(_N):
                v = f[_N - 1]
                for m in range(_N - 2, -1, -1):
                    v = jnp.where(masks[n][m], f[m], v)
                out_b[n, c, hh, sl] = v
        return 0

    lax.fori_loop(0, _GROUPS, group_body, 0)


def _body(feat_hbm, ts_hbm, out_hbm, ts_v, src_v, feat_v, out_v,
          in_sems, out_sems):
    wid = lax.axis_index("s") * 2 + lax.axis_index("c")
    consts = [jnp.full((_L,), m, jnp.int32) for m in range(_N)]

    def unit_body(u, _):
        t = wid * _UNITS_PER_WORKER + u
        b = t // (_HB * (_C // _CQ))
        r = t % (_HB * (_C // _CQ))
        h0 = (r // (_C // _CQ)) * _HH
        c0 = (r % (_C // _CQ)) * _CQ

        pltpu.sync_copy(ts_hbm.at[b, :, pl.ds(h0, _HH), :], ts_v)
        _rank_and_sources(ts_v, src_v, consts)

        def in_copy(cc):
            buf = cc % _NBUF
            return pltpu.make_async_copy(
                feat_hbm.at[b, :, pl.ds(c0 + cc * _CC, _CC),
                            pl.ds(h0, _HH), :],
                feat_v.at[buf], in_sems.at[buf])

        def out_copy(cc):
            buf = cc % _NBUF
            return pltpu.make_async_copy(
                out_v.at[buf],
                out_hbm.at[b, :, pl.ds(c0 + cc * _CC, _CC),
                           pl.ds(h0, _HH), :],
                out_sems.at[buf])

        for cc in range(_DEPTH):
            in_copy(cc).start()
        for cc in range(_CHUNKS):
            buf = cc % _NBUF
            if cc + _DEPTH < _CHUNKS:
                in_copy(cc + _DEPTH).start()
            in_copy(cc).wait()
            if cc >= _NBUF:
                out_copy(cc - _NBUF).wait()
            _permute_chunk(feat_v.at[buf], out_v.at[buf], src_v, consts)
            out_copy(cc).start()
        for cc in range(max(_CHUNKS - _NBUF, 0), _CHUNKS):
            out_copy(cc).wait()
        return 0

    lax.fori_loop(0, _UNITS_PER_WORKER, unit_body, 0)


def kernel(features, timestamps, dim):
    del dim
    mesh = plsc.VectorSubcoreMesh(core_axis_name="c", subcore_axis_name="s")
    run = pl.kernel(
        _body,
        out_type=jax.ShapeDtypeStruct(features.shape, features.dtype),
        mesh=mesh,
        compiler_params=pltpu.CompilerParams(use_tc_tiling_on_sc=False),
        scratch_types=[
            pltpu.VMEM((_N, _HH, _W), jnp.float32),
            pltpu.VMEM((_N, _HH, _W), jnp.int32),
            pltpu.VMEM((_NBUF, _N, _CC, _HH, _W), jnp.float32),
            pltpu.VMEM((_NBUF, _N, _CC, _HH, _W), jnp.float32),
            pltpu.SemaphoreType.DMA((_NBUF,)),
            pltpu.SemaphoreType.DMA((_NBUF,)),
        ],
    )
    return run(features, timestamps)


# speedup vs baseline: 16.3042x; 3.8839x over previous
import functools

import jax
import jax.numpy as jnp
from jax import lax
from jax.experimental import pallas as pl
from jax.experimental.pallas import tpu as pltpu
from jax.experimental.pallas import tpu_sc as plsc

_B, _N, _C, _H, _W = 2, 4, 96, 224, 224
_L = 16
_NUM_WORKERS = 32
_ROWS = _B * _H
_ROWS_PER_WORKER = _ROWS // _NUM_WORKERS
_GROUPS = _ROWS_PER_WORKER * (_W // _L)


def _sc_rank_body(ts_hbm, rank_hbm, ts_v, rank_v):
    wid = lax.axis_index("s") * 2 + lax.axis_index("c")
    consts = [jnp.full((_L,), m, jnp.int32) for m in range(_N)]
    zero, one = consts[0], consts[1]

    row0 = wid * _ROWS_PER_WORKER
    b = row0 // _H
    h0 = row0 % _H

    pltpu.sync_copy(ts_hbm.at[b, :, pl.ds(h0, _ROWS_PER_WORKER), :], ts_v)

    def group_body(fg, _):
        hh = fg // (_W // _L)
        sl = pl.ds((fg % (_W // _L)) * _L, _L)
        t = [ts_v[m, hh, sl] for m in range(_N)]
        for m in range(_N):
            r = zero
            for j in range(_N):
                if j == m:
                    continue
                r = r + jnp.where(t[j] > t[m], one, zero)
                if j < m:
                    r = r + jnp.where(t[j] == t[m], one, zero)
            rank_v[m, hh, sl] = r
        return 0

    lax.fori_loop(0, _GROUPS, group_body, 0)
    pltpu.sync_copy(rank_v,
                    rank_hbm.at[b, :, pl.ds(h0, _ROWS_PER_WORKER), :])


def _sc_ranks(timestamps):
    mesh = plsc.VectorSubcoreMesh(core_axis_name="c", subcore_axis_name="s")
    run = pl.kernel(
        _sc_rank_body,
        out_type=jax.ShapeDtypeStruct((_B, _N, _H, _W), jnp.int32),
        mesh=mesh,
        compiler_params=pltpu.CompilerParams(use_tc_tiling_on_sc=False),
        scratch_types=[
            pltpu.VMEM((_N, _ROWS_PER_WORKER, _W), jnp.float32),
            pltpu.VMEM((_N, _ROWS_PER_WORKER, _W), jnp.int32),
        ],
    )
    return run(timestamps)


_TC_CB = 8


def _tc_body(rank_ref, feat_ref, out_ref):
    ranks = [rank_ref[m] for m in range(_N)]
    for c in range(_TC_CB):
        f = [feat_ref[m, c] for m in range(_N)]
        for n in range(_N):
            v = f[_N - 1]
            for m in range(_N - 2, -1, -1):
                v = jnp.where(ranks[m] == n, f[m], v)
            out_ref[n, c] = v


def _tc_permute(features, ranks):
    b, n, c, h, w = features.shape
    grid = (b, c // _TC_CB)
    return pl.pallas_call(
        _tc_body,
        grid=grid,
        in_specs=[
            pl.BlockSpec((None, n, h, w), lambda bi, ci: (bi, 0, 0, 0)),
            pl.BlockSpec((None, n, _TC_CB, h, w),
                         lambda bi, ci: (bi, 0, ci, 0, 0)),
        ],
        out_specs=pl.BlockSpec((None, n, _TC_CB, h, w),
                               lambda bi, ci: (bi, 0, ci, 0, 0)),
        out_shape=jax.ShapeDtypeStruct(features.shape, features.dtype),
    )(ranks, features)


def kernel(features, timestamps, dim):
    del dim
    return _tc_permute(features, _sc_ranks(timestamps))
